# baseline (speedup 1.0000x reference)
import functools

import jax
import jax.numpy as jnp
from jax import lax
from jax.experimental import pallas as pl
from jax.experimental.pallas import tpu as pltpu
from jax.experimental.pallas import tpu_sc as plsc

_N = 10000
_E_PAD = 172032
_NS = 16
_CH = 896


def _mm_kernel(x_ref, w_ref, am_ref, h_ref, a_ref):
    h = jnp.dot(x_ref[...], w_ref[...], preferred_element_type=jnp.float32)
    m = h.shape[1]
    h_ref[:, :m] = h
    h_ref[:, m:] = jnp.zeros((h.shape[0], 16), jnp.float32)
    a_ref[...] = jnp.dot(h, am_ref[...], preferred_element_type=jnp.float32)


def _matmul_att(x, W, att_src, att_dst):
    n, k = x.shape
    m = W.shape[1]
    am = jnp.concatenate(
        [att_src[:, None], att_dst[:, None], jnp.zeros((m, 126), jnp.float32)],
        axis=1)
    blk = 2000
    h, a = pl.pallas_call(
        _mm_kernel,
        grid=(n // blk,),
        in_specs=[
            pl.BlockSpec((blk, k), lambda i: (i, 0)),
            pl.BlockSpec((k, m), lambda i: (0, 0)),
            pl.BlockSpec((m, 128), lambda i: (0, 0)),
        ],
        out_specs=[
            pl.BlockSpec((blk, m + 16), lambda i: (i, 0)),
            pl.BlockSpec((blk, 128), lambda i: (i, 0)),
        ],
        out_shape=[
            jax.ShapeDtypeStruct((n, m + 16), jnp.float32),
            jax.ShapeDtypeStruct((n, 128), jnp.float32),
        ],
    )(x, W, am)
    return h, a[:, 0], a[:, 1]


def _make_agg(C, RS, npass, relu):
    es = _E_PAD // _NS
    nch = es // _CH
    cvec = _CH // 16
    trows = RS // _NS
    slabs = trows // 16
    cap = _CH + 64
    CW = C + 16
    i32 = jnp.int32
    f32 = jnp.float32

    mesh = plsc.VectorSubcoreMesh(core_axis_name="c", subcore_axis_name="s")
    scratch = [
        pltpu.VMEM((_N,), f32),
        pltpu.VMEM((_N,), f32),
        pltpu.VMEM((_CH,), i32),
        pltpu.VMEM((_CH,), i32),
        pltpu.VMEM((cap,), i32),
        pltpu.VMEM((cap,), i32),
        pltpu.VMEM((cap,), f32),
        pltpu.VMEM((16, CW), f32),
        pltpu.VMEM((16, CW), f32),
        pltpu.VMEM((16, CW), f32),
        pltpu.VMEM((16, CW), f32),
        pltpu.VMEM((C,), f32),
        pltpu.VMEM_SHARED((RS, CW), f32),
        pltpu.SemaphoreType.DMA,
        pltpu.SemaphoreType.DMA,
        pltpu.SemaphoreType.DMA,
        pltpu.SemaphoreType.DMA,
    ]

    @functools.partial(
        pl.kernel,
        out_type=jax.ShapeDtypeStruct((_N, CW), f32),
        mesh=mesh,
        scratch_types=scratch,
        compiler_params=pltpu.CompilerParams(
            needs_layout_passes=False, use_tc_tiling_on_sc=False),
    )
    def agg(asrc_h, adst_h, src_h, dst_h, h_h, bias_h, out_h,
            asrc_v, adst_v, srcc, dstc, st_src, st_dl, st_p,
            gbuf0, gbuf1, sbuf0, sbuf1, bias_v, accum,
            sem0, sem1, ssem0, ssem1):
        cid = lax.axis_index("c")
        sid = lax.axis_index("s")
        lanes = lax.iota(i32, 16)
        pmask = jnp.where(lanes < 2, 1.0, 0.0).astype(f32)
        zvf = jnp.zeros((16,), f32)
        zvi = jnp.zeros((16,), i32)
        gbufs = (gbuf0, gbuf1)
        sbufs = (sbuf0, sbuf1)
        sems = (sem0, sem1)
        ssems = (ssem0, ssem1)

        pltpu.sync_copy(asrc_h, asrc_v)
        pltpu.sync_copy(adst_h, adst_v)
        pltpu.sync_copy(bias_h, bias_v)

        def _tbl_max(tbl):
            def b(i, m):
                return jnp.maximum(m, tbl[pl.ds(i * 16, 16)])
            return jnp.max(lax.fori_loop(0, _N // 16, b, jnp.full((16,), -jnp.inf, f32)))

        shift = _tbl_max(asrc_v) + _tbl_max(adst_v)

        for pz in range(npass):
            lo = (2 * pz + cid) * RS
            hi = lo + RS

            def zbody(j, _):
                for k in range(16):
                    gbuf0.at[k][pl.ds(j * 16, 16)] = zvf
                return 0
            lax.fori_loop(0, CW // 16, zbody, 0)
            def zsbody(s, _):
                row0 = sid * trows + s * 16
                pltpu.sync_copy(gbuf0, accum.at[pl.ds(row0, 16), :])
                return 0
            lax.fori_loop(0, slabs, zsbody, 0)
            plsc.subcore_barrier()

            def chunk_body(ch, _0):
                base = sid * es + ch * _CH
                pltpu.sync_copy(src_h.at[pl.ds(base, _CH)], srcc)
                pltpu.sync_copy(dst_h.at[pl.ds(base, _CH)], dstc)

                def cbody(i, off):
                    s = srcc[pl.ds(i * 16, 16)]
                    d = dstc[pl.ds(i * 16, 16)]
                    m = (d >= lo) & (d < hi)
                    av = plsc.load_gather(asrc_v, [s])
                    bv = plsc.load_gather(adst_v, [jnp.where(m, d, 0)])
                    e = av + bv
                    e = jnp.where(e > 0.0, e, 0.2 * e) - shift
                    p = jnp.exp(e)
                    plsc.store_compressed(st_src.at[pl.ds(off, 16)], s, mask=m)
                    plsc.store_compressed(st_dl.at[pl.ds(off, 16)], d - lo, mask=m)
                    plsc.store_compressed(st_p.at[pl.ds(off + 16, 16)], p, mask=m)
                    return off + jnp.sum(m.astype(i32))

                M = lax.fori_loop(0, cvec, cbody, jnp.zeros((), i32))
                for t in range(3):
                    st_src[pl.ds(M + t * 16, 16)] = zvi
                    st_dl[pl.ds(M + t * 16, 16)] = zvi
                    st_p[pl.ds(M + 16 + t * 16, 16)] = zvf
                mpad = (M // 32 + 1) * 32

                for b in range(2):
                    idx0 = st_src[pl.ds(b * 16, 16)]
                    pltpu.async_copy(h_h.at[idx0], gbufs[b], sems[b])

                def pbody(i2, _):
                    for b in range(2):
                        off = i2 * 32 + b * 16
                        pltpu.make_async_copy(
                            h_h.at[pl.ds(0, 16), :], gbufs[b], sems[b]).wait()

                        @pl.when(i2 >= 1)
                        def _drain():
                            pltpu.make_async_copy(
                                sbufs[b], accum.at[pl.ds(0, 16), :],
                                ssems[b]).wait()

                        dlv = st_dl[pl.ds(off, 16)]
                        pks = [
                            plsc.load_gather(
                                st_p, [jnp.full((16,), off + 16 + k, i32)])
                            for k in range(16)
                        ]

                        def jbody(j, _):
                            for k in range(16):
                                g = gbufs[b].at[k]
                                s = sbufs[b].at[k]
                                s[pl.ds(j * 16, 16)] = g[pl.ds(j * 16, 16)] * pks[k]
                            return 0
                        lax.fori_loop(0, C // 16, jbody, 0)
                        for k in range(16):
                            sbufs[b].at[k][pl.ds(C, 16)] = pks[k] * pmask

                        noff = off + 32

                        @pl.when(noff < mpad)
                        def _fire():
                            idx = st_src[pl.ds(noff, 16)]
                            pltpu.async_copy(h_h.at[idx], gbufs[b], sems[b])

                        pltpu.async_copy(sbufs[b], accum.at[dlv],
                                         ssems[b], add=True)
                    return 0

                lax.fori_loop(0, mpad // 32, pbody, 0)
                for b in range(2):
                    pltpu.make_async_copy(
                        sbufs[b], accum.at[pl.ds(0, 16), :], ssems[b]).wait()
                return 0

            lax.fori_loop(0, nch, chunk_body, 0)
            plsc.subcore_barrier()

            def fin_body(s, _0):
                row0 = sid * trows + s * 16
                gbase = lo + row0

                @pl.when(gbase < _N)
                def _fin():
                    pltpu.sync_copy(accum.at[pl.ds(row0, 16), :], gbuf0)
                    dv = plsc.load_gather(
                        gbuf0, [lanes, jnp.full((16,), C + 1, i32)])
                    st_p[pl.ds(16, 16)] = 1.0 / (dv + 1e-16)
                    rcps = [
                        plsc.load_gather(st_p, [jnp.full((16,), 16 + k, i32)])
                        for k in range(16)
                    ]

                    def fjbody(j, _):
                        bj = bias_v[pl.ds(j * 16, 16)]
                        for k in range(16):
                            r = gbuf0.at[k]
                            v = r[pl.ds(j * 16, 16)] * rcps[k] + bj
                            if relu:
                                v = jnp.maximum(v, 0.0)
                            r[pl.ds(j * 16, 16)] = v
                        return 0
                    lax.fori_loop(0, C // 16, fjbody, 0)
                    pltpu.sync_copy(gbuf0, out_h.at[pl.ds(gbase, 16), :])
                return 0

            lax.fori_loop(0, slabs, fin_body, 0)
            plsc.subcore_barrier()

    return agg


def kernel(x, edge_index, W1, att_src1, att_dst1, b1, W2, att_src2, att_dst2, b2):
    loop = jnp.arange(_N, dtype=edge_index.dtype)
    src = jnp.concatenate([edge_index[0], loop]).astype(jnp.int32)
    dst = jnp.concatenate([edge_index[1], loop]).astype(jnp.int32)
    pad = _E_PAD - src.shape[0]
    src_p = jnp.concatenate([src, jnp.zeros((pad,), jnp.int32)])
    dst_p = jnp.concatenate([dst, jnp.full((pad,), 1 << 30, jnp.int32)])

    h1, as1, ad1 = _matmul_att(x, W1, att_src1, att_dst1)
    agg1 = _make_agg(512, 2048, 3, True)
    o1 = agg1(as1, ad1, src_p, dst_p, h1, b1)[:, :512]
    h2, as2, ad2 = _matmul_att(o1, W2, att_src2, att_dst2)
    agg2 = _make_agg(256, 5120, 1, False)
    return agg2(as2, ad2, src_p, dst_p, h2, b2)[:, :256]


# speedup vs baseline: 1.2242x; 1.2242x over previous
import functools

import jax
import jax.numpy as jnp
from jax import lax
from jax.experimental import pallas as pl
from jax.experimental.pallas import tpu as pltpu
from jax.experimental.pallas import tpu_sc as plsc

_N = 10000
_E_PAD = 172032
_NS = 16
_CH = 896


def _mm_kernel(x_ref, w_ref, am_ref, h_ref, a_ref):
    h = jnp.dot(x_ref[...], w_ref[...], preferred_element_type=jnp.float32)
    m = h.shape[1]
    h_ref[:, :m] = h
    h_ref[:, m:] = jnp.zeros((h.shape[0], 16), jnp.float32)
    a_ref[...] = jnp.dot(h, am_ref[...], preferred_element_type=jnp.float32)


def _matmul_att(x, W, att_src, att_dst):
    n, k = x.shape
    m = W.shape[1]
    am = jnp.concatenate(
        [att_src[:, None], att_dst[:, None], jnp.zeros((m, 126), jnp.float32)],
        axis=1)
    blk = 2000
    h, a = pl.pallas_call(
        _mm_kernel,
        grid=(n // blk,),
        in_specs=[
            pl.BlockSpec((blk, k), lambda i: (i, 0)),
            pl.BlockSpec((k, m), lambda i: (0, 0)),
            pl.BlockSpec((m, 128), lambda i: (0, 0)),
        ],
        out_specs=[
            pl.BlockSpec((blk, m + 16), lambda i: (i, 0)),
            pl.BlockSpec((blk, 128), lambda i: (i, 0)),
        ],
        out_shape=[
            jax.ShapeDtypeStruct((n, m + 16), jnp.float32),
            jax.ShapeDtypeStruct((n, 128), jnp.float32),
        ],
    )(x, W, am)
    return h, a[:, 0], a[:, 1]


def _make_agg(C, RS, npass, relu):
    es = _E_PAD // _NS
    nch = es // _CH
    cvec = _CH // 16
    trows = RS // _NS
    slabs = trows // 16
    cap = _CH + 64
    CW = C + 16
    i32 = jnp.int32
    f32 = jnp.float32

    mesh = plsc.VectorSubcoreMesh(core_axis_name="c", subcore_axis_name="s")
    scratch = [
        pltpu.VMEM((_N,), f32),
        pltpu.VMEM((_N,), f32),
        pltpu.VMEM((_CH,), i32),
        pltpu.VMEM((_CH,), i32),
        pltpu.VMEM((cap,), i32),
        pltpu.VMEM((cap,), i32),
        pltpu.VMEM((cap,), f32),
        pltpu.VMEM((16, CW), f32),
        pltpu.VMEM((16, CW), f32),
        pltpu.VMEM((C,), f32),
        pltpu.VMEM_SHARED((RS, CW), f32),
        pltpu.SemaphoreType.DMA,
        pltpu.SemaphoreType.DMA,
    ]

    @functools.partial(
        pl.kernel,
        out_type=jax.ShapeDtypeStruct((_N, CW), f32),
        mesh=mesh,
        scratch_types=scratch,
        compiler_params=pltpu.CompilerParams(
            needs_layout_passes=False, use_tc_tiling_on_sc=False),
    )
    def agg(asrc_h, adst_h, src_h, dst_h, h_h, bias_h, out_h,
            asrc_v, adst_v, srcc, dstc, st_src, st_dl, st_p,
            gbuf0, gbuf1, bias_v, accum, sem0, sem1):
        cid = lax.axis_index("c")
        sid = lax.axis_index("s")
        lanes = lax.iota(i32, 16)
        pmask = jnp.where(lanes < 2, 1.0, 0.0).astype(f32)
        zvf = jnp.zeros((16,), f32)
        zvi = jnp.zeros((16,), i32)
        gbufs = (gbuf0, gbuf1)
        sems = (sem0, sem1)

        pltpu.sync_copy(asrc_h, asrc_v)
        pltpu.sync_copy(adst_h, adst_v)
        pltpu.sync_copy(bias_h, bias_v)

        def _tbl_max(tbl):
            def b(i, m):
                return jnp.maximum(m, tbl[pl.ds(i * 16, 16)])
            return jnp.max(lax.fori_loop(0, _N // 16, b, jnp.full((16,), -jnp.inf, f32)))

        shift = _tbl_max(asrc_v) + _tbl_max(adst_v)

        for pz in range(npass):
            lo = (2 * pz + cid) * RS
            hi = lo + RS

            def zbody(j, _):
                for k in range(16):
                    gbuf0.at[k][pl.ds(j * 16, 16)] = zvf
                return 0
            lax.fori_loop(0, CW // 16, zbody, 0)
            def zsbody(s, _):
                row0 = sid * trows + s * 16
                pltpu.sync_copy(gbuf0, accum.at[pl.ds(row0, 16), :])
                return 0
            lax.fori_loop(0, slabs, zsbody, 0)
            plsc.subcore_barrier()

            def chunk_body(ch, _0):
                base = sid * es + ch * _CH
                pltpu.sync_copy(src_h.at[pl.ds(base, _CH)], srcc)
                pltpu.sync_copy(dst_h.at[pl.ds(base, _CH)], dstc)

                def cbody(i, off):
                    s = srcc[pl.ds(i * 16, 16)]
                    d = dstc[pl.ds(i * 16, 16)]
                    m = (d >= lo) & (d < hi)
                    av = plsc.load_gather(asrc_v, [s])
                    bv = plsc.load_gather(adst_v, [jnp.where(m, d, 0)])
                    e = av + bv
                    e = jnp.where(e > 0.0, e, 0.2 * e) - shift
                    p = jnp.exp(e)
                    plsc.store_compressed(st_src.at[pl.ds(off, 16)], s, mask=m)
                    plsc.store_compressed(st_dl.at[pl.ds(off, 16)], d - lo, mask=m)
                    plsc.store_compressed(st_p.at[pl.ds(off + 16, 16)], p, mask=m)
                    return off + jnp.sum(m.astype(i32))

                M = lax.fori_loop(0, cvec, cbody, jnp.zeros((), i32))
                for t in range(3):
                    st_src[pl.ds(M + t * 16, 16)] = zvi
                    st_dl[pl.ds(M + t * 16, 16)] = zvi
                    st_p[pl.ds(M + 16 + t * 16, 16)] = zvf
                mpad = (M // 32 + 1) * 32

                for b in range(2):
                    idx0 = st_src[pl.ds(b * 16, 16)]
                    pltpu.async_copy(h_h.at[idx0], gbufs[b], sems[b])

                def pbody(i2, _):
                    for b in range(2):
                        off = i2 * 32 + b * 16
                        pltpu.make_async_copy(
                            h_h.at[pl.ds(0, 16), :], gbufs[b], sems[b]).wait()
                        dlv = st_dl[pl.ds(off, 16)]
                        pks = [
                            plsc.load_gather(
                                st_p, [jnp.full((16,), off + 16 + k, i32)])
                            for k in range(16)
                        ]

                        def jbody(j, _):
                            for k in range(16):
                                r = gbufs[b].at[k]
                                r[pl.ds(j * 16, 16)] = r[pl.ds(j * 16, 16)] * pks[k]
                            return 0
                        lax.fori_loop(0, C // 16, jbody, 0)
                        for k in range(16):
                            gbufs[b].at[k][pl.ds(C, 16)] = pks[k] * pmask
                        pltpu.sync_copy(gbufs[b], accum.at[dlv], add=True)

                        noff = off + 32

                        @pl.when(noff < mpad)
                        def _fire():
                            idx = st_src[pl.ds(noff, 16)]
                            pltpu.async_copy(h_h.at[idx], gbufs[b], sems[b])
                    return 0

                lax.fori_loop(0, mpad // 32, pbody, 0)
                return 0

            lax.fori_loop(0, nch, chunk_body, 0)
            plsc.subcore_barrier()

            def fin_body(s, _0):
                row0 = sid * trows + s * 16
                gbase = lo + row0

                @pl.when(gbase < _N)
                def _fin():
                    pltpu.sync_copy(accum.at[pl.ds(row0, 16), :], gbuf0)
                    dv = plsc.load_gather(
                        gbuf0, [lanes, jnp.full((16,), C + 1, i32)])
                    st_p[pl.ds(16, 16)] = 1.0 / (dv + 1e-16)
                    rcps = [
                        plsc.load_gather(st_p, [jnp.full((16,), 16 + k, i32)])
                        for k in range(16)
                    ]

                    def fjbody(j, _):
                        bj = bias_v[pl.ds(j * 16, 16)]
                        for k in range(16):
                            r = gbuf0.at[k]
                            v = r[pl.ds(j * 16, 16)] * rcps[k] + bj
                            if relu:
                                v = jnp.maximum(v, 0.0)
                            r[pl.ds(j * 16, 16)] = v
                        return 0
                    lax.fori_loop(0, C // 16, fjbody, 0)
                    pltpu.sync_copy(gbuf0, out_h.at[pl.ds(gbase, 16), :])
                return 0

            lax.fori_loop(0, slabs, fin_body, 0)
            plsc.subcore_barrier()

    return agg


def kernel(x, edge_index, W1, att_src1, att_dst1, b1, W2, att_src2, att_dst2, b2):
    loop = jnp.arange(_N, dtype=edge_index.dtype)
    src = jnp.concatenate([edge_index[0], loop]).astype(jnp.int32)
    dst = jnp.concatenate([edge_index[1], loop]).astype(jnp.int32)
    pad = _E_PAD - src.shape[0]
    src_p = jnp.concatenate([src, jnp.zeros((pad,), jnp.int32)])
    dst_p = jnp.concatenate([dst, jnp.full((pad,), 1 << 30, jnp.int32)])

    h1, as1, ad1 = _matmul_att(x, W1, att_src1, att_dst1)
    agg1 = _make_agg(512, 2560, 2, True)
    o1 = agg1(as1, ad1, src_p, dst_p, h1, b1)[:, :512]
    h2, as2, ad2 = _matmul_att(o1, W2, att_src2, att_dst2)
    agg2 = _make_agg(256, 5120, 1, False)
    return agg2(as2, ad2, src_p, dst_p, h2, b2)[:, :256]
